# baseline (speedup 1.0000x reference)
import functools

import jax
import jax.numpy as jnp
from jax import lax
from jax.experimental import pallas as pl
from jax.experimental.pallas import tpu as pltpu
from jax.experimental.pallas import tpu_sc as plsc

N = 10000
D = 128
H = 64
E = 320000

NC = 2
NS = 16
NW = NC * NS

K = 4
STEPS = 20
ROWS_W = STEPS * K
EPW = ROWS_W * 128
EPAD = NW * EPW

RPT = 640
NACC = NS * RPT

_mesh = plsc.VectorSubcoreMesh(
    core_axis_name="c", subcore_axis_name="s", num_cores=NC, num_subcores=NS)


@functools.partial(
    pl.kernel,
    out_type=jax.ShapeDtypeStruct((NC, NACC, 16), jnp.float32),
    mesh=_mesh,
    scratch_types=[
        pltpu.VMEM((ROWS_W, 128), jnp.int32),
        pltpu.VMEM((128, 16), jnp.float32),
        pltpu.VMEM((128, 16), jnp.float32),
        pltpu.VMEM_SHARED((NACC, 16), jnp.float32),
        pltpu.SemaphoreType.DMA,
    ],
    compiler_params=pltpu.CompilerParams(use_tc_tiling_on_sc=False),
)
def _sc_deg(e3_hbm, out_hbm, didx, ones_v, zeros_v, acc, ssem):
    cid = lax.axis_index("c")
    sid = lax.axis_index("s")
    wid = cid * NS + sid

    lane = lax.iota(jnp.int32, 16)
    one_row = jnp.where(lane == 0, 1.0, 0.0).astype(jnp.float32)
    zero_row = jnp.zeros((16,), jnp.float32)

    def _fill(r, carry):
        ones_v[r] = one_row
        zeros_v[r] = zero_row
        return carry

    lax.fori_loop(0, 128, _fill, 0)

    for t in range(RPT // 128):
        pltpu.sync_copy(zeros_v, acc.at[pl.ds(sid * RPT + t * 128, 128)])

    pltpu.sync_copy(e3_hbm.at[1].at[pl.ds(wid * ROWS_W, ROWS_W)], didx)
    plsc.subcore_barrier()

    def _drain(sem):
        pltpu.make_async_copy(out_hbm.at[0].at[pl.ds(0, 128)], zeros_v,
                              sem).wait()

    for j in range(K):
        pltpu.async_copy(ones_v, acc.at[didx.at[j]], ssem, add=True)

    def _step(c, carry):
        for j in range(K):
            pltpu.async_copy(ones_v, acc.at[didx.at[(c + 1) * K + j]], ssem,
                             add=True)
        for j in range(K):
            _drain(ssem)
        return carry

    lax.fori_loop(0, STEPS - 1, _step, 0)
    for j in range(K):
        _drain(ssem)
    plsc.subcore_barrier()

    pltpu.sync_copy(
        acc.at[pl.ds(sid * RPT, RPT)],
        out_hbm.at[cid].at[pl.ds(sid * RPT, RPT)],
    )


@functools.partial(
    pl.kernel,
    out_type=jax.ShapeDtypeStruct((NC, NACC, H), jnp.float32),
    mesh=_mesh,
    scratch_types=[
        pltpu.VMEM((ROWS_W, 128), jnp.int32),
        pltpu.VMEM((ROWS_W, 128), jnp.int32),
        pltpu.VMEM((2, K * 128, H), jnp.float32),
        pltpu.SemaphoreType.DMA,
        pltpu.SemaphoreType.DMA,
        pltpu.SemaphoreType.DMA,
        pltpu.VMEM_SHARED((NACC, H), jnp.float32),
    ],
    compiler_params=pltpu.CompilerParams(use_tc_tiling_on_sc=False),
)
def _sc_agg(table_hbm, e3_hbm, out_hbm, sidx, didx, rows, gsem, sa, sb, acc):
    cid = lax.axis_index("c")
    sid = lax.axis_index("s")
    wid = cid * NS + sid

    zero_row = jnp.zeros((16,), jnp.float32)

    def _fillz(r, carry):
        for q in range(H // 16):
            rows[0, r, pl.ds(q * 16, 16)] = zero_row
        return carry

    lax.fori_loop(0, 128, _fillz, 0)

    for t in range(RPT // 128):
        pltpu.sync_copy(rows.at[0].at[pl.ds(0, 128)],
                        acc.at[pl.ds(sid * RPT + t * 128, 128)])

    pltpu.sync_copy(e3_hbm.at[0].at[pl.ds(wid * ROWS_W, ROWS_W)], sidx)
    pltpu.sync_copy(e3_hbm.at[1].at[pl.ds(wid * ROWS_W, ROWS_W)], didx)
    plsc.subcore_barrier()

    def _drain(sem):
        pltpu.make_async_copy(table_hbm.at[pl.ds(0, 128)],
                              rows.at[0].at[pl.ds(0, 128)], sem).wait()

    def _gather(step, slot, sem):
        for j in range(K):
            pltpu.async_copy(table_hbm.at[sidx.at[step * K + j]],
                             rows.at[slot].at[pl.ds(j * 128, 128)], sem)

    def _scatter(step, slot, sem):
        for j in range(K):
            pltpu.async_copy(rows.at[slot].at[pl.ds(j * 128, 128)],
                             acc.at[didx.at[step * K + j]], sem, add=True)

    _gather(0, 0, gsem)
    for j in range(K):
        pltpu.async_copy(table_hbm.at[pl.ds(0, 128)],
                         rows.at[1].at[pl.ds(j * 128, 128)], sb)

    def _pair(i, carry):
        e = 2 * i
        for j in range(K):
            _drain(gsem)
        _scatter(e, 0, sa)
        for j in range(K):
            _drain(sb)
        _gather(e + 1, 1, gsem)
        for j in range(K):
            _drain(gsem)
        _scatter(e + 1, 1, sb)
        for j in range(K):
            _drain(sa)
        en = jnp.minimum(e + 2, STEPS - 1)
        _gather(en, 0, gsem)
        return carry

    lax.fori_loop(0, STEPS // 2, _pair, 0)
    for j in range(K):
        _drain(sb)
    for j in range(K):
        _drain(gsem)
    plsc.subcore_barrier()

    pltpu.sync_copy(
        acc.at[pl.ds(sid * RPT, RPT)],
        out_hbm.at[cid].at[pl.ds(sid * RPT, RPT)],
    )


TBLK = N // 5
TGRID = 5


def _dinv_of(degp_ref):
    deg = degp_ref[0][:, 0:1] + degp_ref[1][:, 0:1] + 1.0
    return lax.rsqrt(deg)


def _tc_mm_body(x_ref, w_ref, o_ref):
    o_ref[...] = jnp.dot(x_ref[...], w_ref[...],
                         preferred_element_type=jnp.float32)


_tc_mm = pl.pallas_call(
    _tc_mm_body,
    grid=(TGRID,),
    in_specs=[
        pl.BlockSpec((TBLK, D), lambda i: (i, 0)),
        pl.BlockSpec((D, H), lambda i: (0, 0)),
    ],
    out_specs=pl.BlockSpec((TBLK, H), lambda i: (i, 0)),
    out_shape=jax.ShapeDtypeStruct((N, H), jnp.float32),
)


def _tc_scale_body(h_ref, degp_ref, o_ref):
    o_ref[...] = h_ref[...] * _dinv_of(degp_ref)


_tc_scale = pl.pallas_call(
    _tc_scale_body,
    grid=(TGRID,),
    in_specs=[
        pl.BlockSpec((TBLK, H), lambda i: (i, 0)),
        pl.BlockSpec((NC, TBLK, 16), lambda i: (0, i, 0)),
    ],
    out_specs=pl.BlockSpec((TBLK, H), lambda i: (i, 0)),
    out_shape=jax.ShapeDtypeStruct((N, H), jnp.float32),
)


def _tc_relu_body(aggp_ref, degp_ref, hs_ref, b_ref, o_ref):
    dinv = _dinv_of(degp_ref)
    agg = aggp_ref[0] + aggp_ref[1] + hs_ref[...]
    s = agg * dinv + b_ref[0:1, :]
    o_ref[...] = jnp.maximum(s, 0.0) * dinv


_tc_relu = pl.pallas_call(
    _tc_relu_body,
    grid=(TGRID,),
    in_specs=[
        pl.BlockSpec((NC, TBLK, H), lambda i: (0, i, 0)),
        pl.BlockSpec((NC, TBLK, 16), lambda i: (0, i, 0)),
        pl.BlockSpec((TBLK, H), lambda i: (i, 0)),
        pl.BlockSpec((8, H), lambda i: (0, 0)),
    ],
    out_specs=pl.BlockSpec((TBLK, H), lambda i: (i, 0)),
    out_shape=jax.ShapeDtypeStruct((N, H), jnp.float32),
)


def _tc_out_body(aggp_ref, degp_ref, rs_ref, w_ref, b_ref, o_ref):
    dinv = _dinv_of(degp_ref)
    a = (aggp_ref[0] + aggp_ref[1] + rs_ref[...]) * dinv
    z = jnp.dot(a, w_ref[...], preferred_element_type=jnp.float32) + b_ref[0:1, :]
    m = jnp.max(z, axis=1, keepdims=True)
    e = jnp.exp(z - m)
    s = jnp.sum(e, axis=1, keepdims=True)
    o_ref[...] = z - m - jnp.log(s)


_tc_out = pl.pallas_call(
    _tc_out_body,
    grid=(TGRID,),
    in_specs=[
        pl.BlockSpec((NC, TBLK, H), lambda i: (0, i, 0)),
        pl.BlockSpec((NC, TBLK, 16), lambda i: (0, i, 0)),
        pl.BlockSpec((TBLK, H), lambda i: (i, 0)),
        pl.BlockSpec((H, 3), lambda i: (0, 0)),
        pl.BlockSpec((8, 3), lambda i: (0, 0)),
    ],
    out_specs=pl.BlockSpec((TBLK, 3), lambda i: (i, 0)),
    out_shape=jax.ShapeDtypeStruct((N, 3), jnp.float32),
)


def kernel(x, edge_index, W1, b1, W2, b2):
    pad = EPAD - E
    pad_ar = jnp.arange(pad, dtype=jnp.int32)
    pad2 = jnp.stack([pad_ar & 8191, N + (pad_ar & 127)])
    e3 = jnp.concatenate([edge_index, pad2], axis=1).reshape(2, NW * ROWS_W, 128)

    h = _tc_mm(x, W1)
    degp = _sc_deg(e3)

    hs = _tc_scale(h, degp)
    agg1 = _sc_agg(hs, e3)

    b1r = jnp.broadcast_to(b1[None, :], (8, H))
    rs = _tc_relu(agg1, degp, hs, b1r)
    agg2 = _sc_agg(rs, e3)

    b2r = jnp.broadcast_to(b2[None, :], (8, 3))
    return _tc_out(agg2, degp, rs, W2, b2r)


# speedup vs baseline: 1.0712x; 1.0712x over previous
import functools

import jax
import jax.numpy as jnp
from jax import lax
from jax.experimental import pallas as pl
from jax.experimental.pallas import tpu as pltpu
from jax.experimental.pallas import tpu_sc as plsc

N = 10000
D = 128
H = 64
E = 320000

NC = 2
NS = 16
NW = NC * NS

K = 4
STEPS = 20
ROWS_W = STEPS * K
EPW = ROWS_W * 128
EPAD = NW * EPW

RPT = 640
NACC = NS * RPT

_mesh = plsc.VectorSubcoreMesh(
    core_axis_name="c", subcore_axis_name="s", num_cores=NC, num_subcores=NS)


@functools.partial(
    pl.kernel,
    out_type=jax.ShapeDtypeStruct((NC, NACC, 16), jnp.float32),
    mesh=_mesh,
    scratch_types=[
        pltpu.VMEM((ROWS_W, 128), jnp.int32),
        pltpu.VMEM((128, 16), jnp.float32),
        pltpu.VMEM((128, 16), jnp.float32),
        pltpu.VMEM_SHARED((NACC, 16), jnp.float32),
        pltpu.SemaphoreType.DMA,
    ],
    compiler_params=pltpu.CompilerParams(use_tc_tiling_on_sc=False),
)
def _sc_deg(e3_hbm, out_hbm, didx, ones_v, zeros_v, acc, ssem):
    cid = lax.axis_index("c")
    sid = lax.axis_index("s")
    wid = cid * NS + sid

    lane = lax.iota(jnp.int32, 16)
    one_row = jnp.where(lane == 0, 1.0, 0.0).astype(jnp.float32)
    zero_row = jnp.zeros((16,), jnp.float32)

    def _fill(r, carry):
        ones_v[r] = one_row
        zeros_v[r] = zero_row
        return carry

    lax.fori_loop(0, 128, _fill, 0)

    for t in range(RPT // 128):
        pltpu.sync_copy(zeros_v, acc.at[pl.ds(sid * RPT + t * 128, 128)])

    pltpu.sync_copy(e3_hbm.at[1].at[pl.ds(wid * ROWS_W, ROWS_W)], didx)
    plsc.subcore_barrier()

    def _drain(sem):
        pltpu.make_async_copy(out_hbm.at[0].at[pl.ds(0, 128)], zeros_v,
                              sem).wait()

    for j in range(K):
        pltpu.async_copy(ones_v, acc.at[didx.at[j]], ssem, add=True)

    def _step(c, carry):
        for j in range(K):
            pltpu.async_copy(ones_v, acc.at[didx.at[(c + 1) * K + j]], ssem,
                             add=True)
        for j in range(K):
            _drain(ssem)
        return carry

    lax.fori_loop(0, STEPS - 1, _step, 0)
    for j in range(K):
        _drain(ssem)
    plsc.subcore_barrier()

    pltpu.sync_copy(
        acc.at[pl.ds(sid * RPT, RPT)],
        out_hbm.at[cid].at[pl.ds(sid * RPT, RPT)],
    )


@functools.partial(
    pl.kernel,
    out_type=jax.ShapeDtypeStruct((NC, NACC, H), jnp.float32),
    mesh=_mesh,
    scratch_types=[
        pltpu.VMEM((ROWS_W, 128), jnp.int32),
        pltpu.VMEM((ROWS_W, 128), jnp.int32),
        pltpu.VMEM((2, K * 128, H), jnp.float32),
        pltpu.SemaphoreType.DMA,
        pltpu.SemaphoreType.DMA,
        pltpu.VMEM_SHARED((NACC, H), jnp.float32),
    ],
    compiler_params=pltpu.CompilerParams(use_tc_tiling_on_sc=False),
)
def _sc_agg(table_hbm, e3_hbm, out_hbm, sidx, didx, rows, gsem, sa, acc):
    cid = lax.axis_index("c")
    sid = lax.axis_index("s")
    wid = cid * NS + sid

    zero_row = jnp.zeros((16,), jnp.float32)

    def _fillz(r, carry):
        for q in range(H // 16):
            rows[0, r, pl.ds(q * 16, 16)] = zero_row
        return carry

    lax.fori_loop(0, 128, _fillz, 0)

    for t in range(RPT // 128):
        pltpu.sync_copy(rows.at[0].at[pl.ds(0, 128)],
                        acc.at[pl.ds(sid * RPT + t * 128, 128)])

    pltpu.sync_copy(e3_hbm.at[0].at[pl.ds(wid * ROWS_W, ROWS_W)], sidx)
    pltpu.sync_copy(e3_hbm.at[1].at[pl.ds(wid * ROWS_W, ROWS_W)], didx)
    plsc.subcore_barrier()

    def _drain(sem):
        pltpu.make_async_copy(table_hbm.at[pl.ds(0, 128)],
                              rows.at[0].at[pl.ds(0, 128)], sem).wait()

    for j in range(K):
        pltpu.async_copy(table_hbm.at[sidx.at[j]],
                         rows.at[0].at[pl.ds(j * 128, 128)], gsem)

    def _step(c, carry):
        slot = lax.rem(c, 2)
        rv = rows.at[slot]
        rg = rows.at[1 - slot]
        for j in range(K):
            _drain(gsem)
        for j in range(K):
            pltpu.async_copy(rv.at[pl.ds(j * 128, 128)],
                             acc.at[didx.at[c * K + j]], sa, add=True)
        cn = jnp.minimum(c + 1, STEPS - 1)
        for j in range(K):
            pltpu.async_copy(table_hbm.at[sidx.at[cn * K + j]],
                             rg.at[pl.ds(j * 128, 128)], gsem)
        for j in range(K):
            _drain(sa)
        return carry

    lax.fori_loop(0, STEPS, _step, 0)
    for j in range(K):
        _drain(gsem)
    plsc.subcore_barrier()

    pltpu.sync_copy(
        acc.at[pl.ds(sid * RPT, RPT)],
        out_hbm.at[cid].at[pl.ds(sid * RPT, RPT)],
    )


TBLK = N // 5
TGRID = 5


def _dinv_of(degp_ref):
    deg = degp_ref[0][:, 0:1] + degp_ref[1][:, 0:1] + 1.0
    return lax.rsqrt(deg)


def _tc_mm_body(x_ref, w_ref, o_ref):
    o_ref[...] = jnp.dot(x_ref[...], w_ref[...],
                         preferred_element_type=jnp.float32)


_tc_mm = pl.pallas_call(
    _tc_mm_body,
    grid=(TGRID,),
    in_specs=[
        pl.BlockSpec((TBLK, D), lambda i: (i, 0)),
        pl.BlockSpec((D, H), lambda i: (0, 0)),
    ],
    out_specs=pl.BlockSpec((TBLK, H), lambda i: (i, 0)),
    out_shape=jax.ShapeDtypeStruct((N, H), jnp.float32),
)


def _tc_scale_body(h_ref, degp_ref, o_ref):
    o_ref[...] = h_ref[...] * _dinv_of(degp_ref)


_tc_scale = pl.pallas_call(
    _tc_scale_body,
    grid=(TGRID,),
    in_specs=[
        pl.BlockSpec((TBLK, H), lambda i: (i, 0)),
        pl.BlockSpec((NC, TBLK, 16), lambda i: (0, i, 0)),
    ],
    out_specs=pl.BlockSpec((TBLK, H), lambda i: (i, 0)),
    out_shape=jax.ShapeDtypeStruct((N, H), jnp.float32),
)


def _tc_relu_body(aggp_ref, degp_ref, hs_ref, b_ref, o_ref):
    dinv = _dinv_of(degp_ref)
    agg = aggp_ref[0] + aggp_ref[1] + hs_ref[...]
    s = agg * dinv + b_ref[0:1, :]
    o_ref[...] = jnp.maximum(s, 0.0) * dinv


_tc_relu = pl.pallas_call(
    _tc_relu_body,
    grid=(TGRID,),
    in_specs=[
        pl.BlockSpec((NC, TBLK, H), lambda i: (0, i, 0)),
        pl.BlockSpec((NC, TBLK, 16), lambda i: (0, i, 0)),
        pl.BlockSpec((TBLK, H), lambda i: (i, 0)),
        pl.BlockSpec((8, H), lambda i: (0, 0)),
    ],
    out_specs=pl.BlockSpec((TBLK, H), lambda i: (i, 0)),
    out_shape=jax.ShapeDtypeStruct((N, H), jnp.float32),
)


def _tc_out_body(aggp_ref, degp_ref, rs_ref, w_ref, b_ref, o_ref):
    dinv = _dinv_of(degp_ref)
    a = (aggp_ref[0] + aggp_ref[1] + rs_ref[...]) * dinv
    z = jnp.dot(a, w_ref[...], preferred_element_type=jnp.float32) + b_ref[0:1, :]
    m = jnp.max(z, axis=1, keepdims=True)
    e = jnp.exp(z - m)
    s = jnp.sum(e, axis=1, keepdims=True)
    o_ref[...] = z - m - jnp.log(s)


_tc_out = pl.pallas_call(
    _tc_out_body,
    grid=(TGRID,),
    in_specs=[
        pl.BlockSpec((NC, TBLK, H), lambda i: (0, i, 0)),
        pl.BlockSpec((NC, TBLK, 16), lambda i: (0, i, 0)),
        pl.BlockSpec((TBLK, H), lambda i: (i, 0)),
        pl.BlockSpec((H, 3), lambda i: (0, 0)),
        pl.BlockSpec((8, 3), lambda i: (0, 0)),
    ],
    out_specs=pl.BlockSpec((TBLK, 3), lambda i: (i, 0)),
    out_shape=jax.ShapeDtypeStruct((N, 3), jnp.float32),
)


def kernel(x, edge_index, W1, b1, W2, b2):
    pad = EPAD - E
    pad_ar = jnp.arange(pad, dtype=jnp.int32)
    pad2 = jnp.stack([pad_ar & 8191, N + (pad_ar & 127)])
    e3 = jnp.concatenate([edge_index, pad2], axis=1).reshape(2, NW * ROWS_W, 128)

    h = _tc_mm(x, W1)
    degp = _sc_deg(e3)

    hs = _tc_scale(h, degp)
    agg1 = _sc_agg(hs, e3)

    b1r = jnp.broadcast_to(b1[None, :], (8, H))
    rs = _tc_relu(agg1, degp, hs, b1r)
    agg2 = _sc_agg(rs, e3)

    b2r = jnp.broadcast_to(b2[None, :], (8, 3))
    return _tc_out(agg2, degp, rs, W2, b2r)
